# baseline (speedup 1.0000x reference)
import functools

import jax
import jax.numpy as jnp
from jax import lax
from jax.experimental import pallas as pl
from jax.experimental.pallas import tpu as pltpu
from jax.experimental.pallas import tpu_sc as plsc

B, H, W = 16, 512, 512
NCORE, NSUB, L = 2, 16, 16

HSC = 256
NCH = (HSC // 2) // 8
GROUPS = (8 * W) // L
NACC = 18
ACCW = NACC * L


def _sc_body(xt_hbm, pt_hbm, tt_hbm, out_hbm,
             xb0, xb1, pb0, pb1, tb0, tb1, accb, sems):
    cid = lax.axis_index("c")
    sid = lax.axis_index("s")
    h_base = cid * (HSC // 2)

    bufs = ((xb0, pb0, tb0), (xb1, pb1, tb1))

    def start(i, k):
        xb, pb, tb = bufs[k]
        h0 = h_base + i * 8
        for m in range(2):
            pltpu.async_copy(
                xt_hbm.at[sid, pl.ds(h0, 8), 4 + m, :], xb.at[m], sems.at[k])
        for ch in range(3):
            pltpu.async_copy(
                pt_hbm.at[sid, ch, pl.ds(h0, 8), :], pb.at[ch], sems.at[k])
            pltpu.async_copy(
                tt_hbm.at[sid, ch, pl.ds(h0, 8), :], tb.at[ch], sems.at[k])

    def waitslot(k):
        xb, pb, tb = bufs[k]
        for m in range(2):
            pltpu.make_async_copy(
                xt_hbm.at[0, pl.ds(0, 8), 4, :], xb.at[m], sems.at[k]).wait()
        for ch in range(3):
            pltpu.make_async_copy(
                pt_hbm.at[0, 0, pl.ds(0, 8), :], pb.at[ch], sems.at[k]).wait()
            pltpu.make_async_copy(
                tt_hbm.at[0, 0, pl.ds(0, 8), :], tb.at[ch], sems.at[k]).wait()

    one = jnp.full((L,), 1.0, jnp.float32)
    zero = jnp.zeros((L,), jnp.float32)

    def compute(k, acc):
        xb, pb, tb = bufs[k]

        def gbody(g, a):
            r = g >> 5
            w0 = (g & 31) * L
            x4 = xb[0, r, pl.ds(w0, L)]
            x5 = xb[1, r, pl.ds(w0, L)]
            a = list(a)
            for c in range(3):
                p = pb[c, r, pl.ds(w0, L)]
                t = tb[c, r, pl.ds(w0, L)]
                nz = p != 0.0
                a[0 + c] = a[0 + c] + jnp.where(nz, x4, zero)
                a[3 + c] = a[3 + c] + jnp.where(nz, x5, zero)
                a[6 + c] = a[6 + c] + x4 * p
                a[9 + c] = a[9 + c] + x5 * p
                a[12 + c] = a[12 + c] + x4 * t
                a[15 + c] = a[15 + c] + x5 * t
            return tuple(a)

        return lax.fori_loop(0, GROUPS, gbody, acc)

    start(0, 0)
    start(1, 1)
    acc = tuple(jnp.zeros((L,), jnp.float32) for _ in range(NACC))

    def outer(j, acc):
        i0 = 2 * j

        waitslot(0)
        acc = compute(0, acc)

        @pl.when(i0 + 2 < NCH)
        def _():
            start(i0 + 2, 0)

        waitslot(1)
        acc = compute(1, acc)

        @pl.when(i0 + 3 < NCH)
        def _():
            start(i0 + 3, 1)

        return acc

    acc = lax.fori_loop(0, NCH // 2, outer, acc)

    for k in range(NACC):
        accb[pl.ds(k * L, L)] = acc[k]
    wid = cid * NSUB + sid
    pltpu.sync_copy(accb, out_hbm.at[wid])


@functools.cache
def _sc_partials():
    return pl.kernel(
        _sc_body,
        out_type=jax.ShapeDtypeStruct((NCORE * NSUB, ACCW), jnp.float32),
        mesh=plsc.VectorSubcoreMesh(
            core_axis_name="c", subcore_axis_name="s",
            num_cores=NCORE, num_subcores=NSUB,
        ),
        scratch_types=[
            pltpu.VMEM((2, 8, W), jnp.float32),
            pltpu.VMEM((2, 8, W), jnp.float32),
            pltpu.VMEM((3, 8, W), jnp.float32),
            pltpu.VMEM((3, 8, W), jnp.float32),
            pltpu.VMEM((3, 8, W), jnp.float32),
            pltpu.VMEM((3, 8, W), jnp.float32),
            pltpu.VMEM((ACCW,), jnp.float32),
            pltpu.SemaphoreType.DMA((2,)),
        ],
        compiler_params=pltpu.CompilerParams(needs_layout_passes=False),
    )


def _tc_dense_body(p_ref, t_ref, x4_ref, x5_ref, out_ref):
    zt = jnp.zeros((8, 128), jnp.float32)
    ot = jnp.ones((8, 128), jnp.float32)
    for c in range(3):
        sq = cn = qa = pa = ta = zt
        for s in range(H // 8):
            for k in range(4):
                p = p_ref[0, c, pl.ds(s * 8, 8), pl.ds(k * 128, 128)]
                t = t_ref[0, c, pl.ds(s * 8, 8), pl.ds(k * 128, 128)]
                m = t != 0.0
                d = p - t
                sq = sq + jnp.where(m, d * d, zt)
                cn = cn + jnp.where(m, ot, zt)
                qa = qa + jnp.where(p != 0.0, ot, zt)
                pa = pa + p
                ta = ta + t

        q4 = q5 = p4 = p5 = t4 = t5 = zt
        for s in range((H - HSC) // 8):
            for k in range(4):
                p = p_ref[0, c, pl.ds(HSC + s * 8, 8), pl.ds(k * 128, 128)]
                t = t_ref[0, c, pl.ds(HSC + s * 8, 8), pl.ds(k * 128, 128)]
                x4 = x4_ref[0, pl.ds(s * 8, 8), pl.ds(k * 128, 128)]
                x5 = x5_ref[0, pl.ds(s * 8, 8), pl.ds(k * 128, 128)]
                nz = p != 0.0
                q4 = q4 + jnp.where(nz, x4, zt)
                q5 = q5 + jnp.where(nz, x5, zt)
                p4 = p4 + x4 * p
                p5 = p5 + x5 * p
                t4 = t4 + x4 * t
                t5 = t5 + x5 * t

        vals = (jnp.sum(qa), jnp.sum(pa), jnp.sum(ta), jnp.sum(sq),
                jnp.sum(cn), jnp.sum(q4), jnp.sum(q5), jnp.sum(p4),
                jnp.sum(p5), jnp.sum(t4), jnp.sum(t5))
        for i, v in enumerate(vals):
            out_ref[0, c, i] = v
        for i in range(11, 16):
            out_ref[0, c, i] = jnp.float32(0.0)


@functools.cache
def _tc_dense():
    return pl.pallas_call(
        _tc_dense_body,
        grid=(B,),
        in_specs=[
            pl.BlockSpec((1, 3, H, W), lambda b: (b, 0, 0, 0)),
            pl.BlockSpec((1, 3, H, W), lambda b: (b, 0, 0, 0)),
            pl.BlockSpec((1, H - HSC, W), lambda b: (b, 0, 0)),
            pl.BlockSpec((1, H - HSC, W), lambda b: (b, 0, 0)),
        ],
        out_specs=pl.BlockSpec((1, 3, 16), lambda b: (b, 0, 0),
                               memory_space=pltpu.SMEM),
        out_shape=jax.ShapeDtypeStruct((B, 3, 16), jnp.float32),
    )


def _epilogue_body(part_ref, dense_ref, out_ref):
    pt = part_ref[...]
    comb = pt[0:NSUB, :] + pt[NSUB:2 * NSUB, :]
    dn = dense_ref[...]

    def grp(k):
        return jnp.sum(comb[:, k * L:(k + 1) * L], axis=1)

    loss = jnp.float32(0.0)
    for c in range(3):
        sq = jnp.sum(dn[:, c, 3])
        cn = jnp.sum(dn[:, c, 4])
        loss = loss + jnp.where(cn > 0, sq / jnp.where(cn > 0, cn, 1.0), 0.0)
        qa, pa, ta = dn[:, c, 0], dn[:, c, 1], dn[:, c, 2]
        q4, q5 = grp(0 + c) + dn[:, c, 5], grp(3 + c) + dn[:, c, 6]
        p4, p5 = grp(6 + c) + dn[:, c, 7], grp(9 + c) + dn[:, c, 8]
        t4, t5 = grp(12 + c) + dn[:, c, 9], grp(15 + c) + dn[:, c, 10]
        q3, p3, t3 = qa - q4 - q5, pa - p4 - p5, ta - t4 - t5
        for qm, ps, ts in ((q3, p3, t3), (q4, p4, t4), (q5, p5, t5)):
            den = jnp.where(qm != 0, qm, 1.0)
            pmean = jnp.where(qm != 0, ps / den, 0.0)
            tmean = jnp.where(qm != 0, ts / den, 0.0)
            loss = loss + jnp.mean((pmean - tmean) ** 2)
    out_ref[0, 0] = loss


def kernel(x, y_pred, y_true):
    xt = jnp.transpose(x, (0, 1, 3, 2))
    pt = jnp.transpose(y_pred, (0, 3, 1, 2))
    tt = jnp.transpose(y_true, (0, 3, 1, 2))
    partials = _sc_partials()(xt, pt, tt)
    x4a = xt[:, HSC:, 4, :]
    x5a = xt[:, HSC:, 5, :]
    dense = _tc_dense()(pt, tt, x4a, x5a)
    res = pl.pallas_call(
        _epilogue_body,
        out_shape=jax.ShapeDtypeStruct((1, 1), jnp.float32),
        out_specs=pl.BlockSpec(memory_space=pltpu.SMEM),
    )(partials, dense)
    return res[0, 0]


# speedup vs baseline: 1.7221x; 1.7221x over previous
import functools

import jax
import jax.numpy as jnp
from jax import lax
from jax.experimental import pallas as pl
from jax.experimental.pallas import tpu as pltpu
from jax.experimental.pallas import tpu_sc as plsc

B, H, W = 16, 512, 512
NCORE, NSUB, L = 2, 16, 16

HSC = 512
NCH = (HSC // 2) // 8
GROUPS = (8 * W) // L
NACC = 18
ACCW = NACC * L


def _sc_body(xt_hbm, pt_hbm, tt_hbm, out_hbm,
             xb0, xb1, pb0, pb1, tb0, tb1, accb, sems):
    cid = lax.axis_index("c")
    sid = lax.axis_index("s")
    h_base = cid * (HSC // 2)

    bufs = ((xb0, pb0, tb0), (xb1, pb1, tb1))

    def start(i, k):
        xb, pb, tb = bufs[k]
        h0 = h_base + i * 8
        for m in range(2):
            pltpu.async_copy(
                xt_hbm.at[sid, pl.ds(h0, 8), 4 + m, :], xb.at[m], sems.at[k])
        for ch in range(3):
            pltpu.async_copy(
                pt_hbm.at[sid, ch, pl.ds(h0, 8), :], pb.at[ch], sems.at[k])
            pltpu.async_copy(
                tt_hbm.at[sid, ch, pl.ds(h0, 8), :], tb.at[ch], sems.at[k])

    def waitslot(k):
        xb, pb, tb = bufs[k]
        for m in range(2):
            pltpu.make_async_copy(
                xt_hbm.at[0, pl.ds(0, 8), 4, :], xb.at[m], sems.at[k]).wait()
        for ch in range(3):
            pltpu.make_async_copy(
                pt_hbm.at[0, 0, pl.ds(0, 8), :], pb.at[ch], sems.at[k]).wait()
            pltpu.make_async_copy(
                tt_hbm.at[0, 0, pl.ds(0, 8), :], tb.at[ch], sems.at[k]).wait()

    one = jnp.full((L,), 1.0, jnp.float32)
    zero = jnp.zeros((L,), jnp.float32)

    def compute(k, acc):
        xb, pb, tb = bufs[k]

        def gbody(g, a):
            r = g >> 5
            w0 = (g & 31) * L
            x4 = xb[0, r, pl.ds(w0, L)]
            x5 = xb[1, r, pl.ds(w0, L)]
            a = list(a)
            for c in range(3):
                p = pb[c, r, pl.ds(w0, L)]
                t = tb[c, r, pl.ds(w0, L)]
                nz = p != 0.0
                a[0 + c] = a[0 + c] + jnp.where(nz, x4, zero)
                a[3 + c] = a[3 + c] + jnp.where(nz, x5, zero)
                a[6 + c] = a[6 + c] + x4 * p
                a[9 + c] = a[9 + c] + x5 * p
                a[12 + c] = a[12 + c] + x4 * t
                a[15 + c] = a[15 + c] + x5 * t
            return tuple(a)

        return lax.fori_loop(0, GROUPS, gbody, acc)

    start(0, 0)
    start(1, 1)
    acc = tuple(jnp.zeros((L,), jnp.float32) for _ in range(NACC))

    def outer(j, acc):
        i0 = 2 * j

        waitslot(0)
        acc = compute(0, acc)

        @pl.when(i0 + 2 < NCH)
        def _():
            start(i0 + 2, 0)

        waitslot(1)
        acc = compute(1, acc)

        @pl.when(i0 + 3 < NCH)
        def _():
            start(i0 + 3, 1)

        return acc

    acc = lax.fori_loop(0, NCH // 2, outer, acc)

    for k in range(NACC):
        accb[pl.ds(k * L, L)] = acc[k]
    wid = cid * NSUB + sid
    pltpu.sync_copy(accb, out_hbm.at[wid])


@functools.cache
def _sc_partials():
    return pl.kernel(
        _sc_body,
        out_type=jax.ShapeDtypeStruct((NCORE * NSUB, ACCW), jnp.float32),
        mesh=plsc.VectorSubcoreMesh(
            core_axis_name="c", subcore_axis_name="s",
            num_cores=NCORE, num_subcores=NSUB,
        ),
        scratch_types=[
            pltpu.VMEM((2, 8, W), jnp.float32),
            pltpu.VMEM((2, 8, W), jnp.float32),
            pltpu.VMEM((3, 8, W), jnp.float32),
            pltpu.VMEM((3, 8, W), jnp.float32),
            pltpu.VMEM((3, 8, W), jnp.float32),
            pltpu.VMEM((3, 8, W), jnp.float32),
            pltpu.VMEM((ACCW,), jnp.float32),
            pltpu.SemaphoreType.DMA((2,)),
        ],
        compiler_params=pltpu.CompilerParams(needs_layout_passes=False),
    )


def _tc_dense_body(p_ref, t_ref, out_ref):
    zt = jnp.zeros((8, 128), jnp.float32)
    ot = jnp.ones((8, 128), jnp.float32)
    for c in range(3):
        sq = cn = qa = pa = ta = zt
        for s in range(H // 8):
            for k in range(4):
                p = p_ref[0, c, pl.ds(s * 8, 8), pl.ds(k * 128, 128)]
                t = t_ref[0, c, pl.ds(s * 8, 8), pl.ds(k * 128, 128)]
                m = t != 0.0
                d = p - t
                sq = sq + jnp.where(m, d * d, zt)
                cn = cn + jnp.where(m, ot, zt)
                qa = qa + jnp.where(p != 0.0, ot, zt)
                pa = pa + p
                ta = ta + t

        zero = jnp.float32(0.0)
        vals = (jnp.sum(qa), jnp.sum(pa), jnp.sum(ta), jnp.sum(sq),
                jnp.sum(cn), zero, zero, zero, zero, zero, zero)
        for i, v in enumerate(vals):
            out_ref[0, c, i] = v
        for i in range(11, 16):
            out_ref[0, c, i] = jnp.float32(0.0)


@functools.cache
def _tc_dense():
    return pl.pallas_call(
        _tc_dense_body,
        grid=(B,),
        in_specs=[
            pl.BlockSpec((1, 3, H, W), lambda b: (b, 0, 0, 0)),
            pl.BlockSpec((1, 3, H, W), lambda b: (b, 0, 0, 0)),
        ],
        out_specs=pl.BlockSpec((1, 3, 16), lambda b: (b, 0, 0),
                               memory_space=pltpu.SMEM),
        out_shape=jax.ShapeDtypeStruct((B, 3, 16), jnp.float32),
    )


def _epilogue_body(part_ref, dense_ref, out_ref):
    pt = part_ref[...]
    comb = pt[0:NSUB, :] + pt[NSUB:2 * NSUB, :]
    dn = dense_ref[...]

    def grp(k):
        return jnp.sum(comb[:, k * L:(k + 1) * L], axis=1)

    loss = jnp.float32(0.0)
    for c in range(3):
        sq = jnp.sum(dn[:, c, 3])
        cn = jnp.sum(dn[:, c, 4])
        loss = loss + jnp.where(cn > 0, sq / jnp.where(cn > 0, cn, 1.0), 0.0)
        qa, pa, ta = dn[:, c, 0], dn[:, c, 1], dn[:, c, 2]
        q4, q5 = grp(0 + c) + dn[:, c, 5], grp(3 + c) + dn[:, c, 6]
        p4, p5 = grp(6 + c) + dn[:, c, 7], grp(9 + c) + dn[:, c, 8]
        t4, t5 = grp(12 + c) + dn[:, c, 9], grp(15 + c) + dn[:, c, 10]
        q3, p3, t3 = qa - q4 - q5, pa - p4 - p5, ta - t4 - t5
        for qm, ps, ts in ((q3, p3, t3), (q4, p4, t4), (q5, p5, t5)):
            den = jnp.where(qm != 0, qm, 1.0)
            pmean = jnp.where(qm != 0, ps / den, 0.0)
            tmean = jnp.where(qm != 0, ts / den, 0.0)
            loss = loss + jnp.mean((pmean - tmean) ** 2)
    out_ref[0, 0] = loss


def kernel(x, y_pred, y_true):
    xt = jnp.transpose(x, (0, 1, 3, 2))
    pt = jnp.transpose(y_pred, (0, 3, 1, 2))
    tt = jnp.transpose(y_true, (0, 3, 1, 2))
    partials = _sc_partials()(xt, pt, tt)
    dense = _tc_dense()(pt, tt)
    res = pl.pallas_call(
        _epilogue_body,
        out_shape=jax.ShapeDtypeStruct((1, 1), jnp.float32),
        out_specs=pl.BlockSpec(memory_space=pltpu.SMEM),
    )(partials, dense)
    return res[0, 0]
